# Initial kernel scaffold; baseline (speedup 1.0000x reference)
import jax
import jax.numpy as jnp
from jax.experimental import pallas as pl


def kernel(x, edge_index, W1, b1, W2, b2, Wih, Whh, bih, bhh, Wfc, bfc):
    raise NotImplementedError("write your pallas kernel here")


# speedup vs baseline: 13.3178x; 13.3178x over previous
import functools

import jax
import jax.numpy as jnp
from jax import lax
from jax.experimental import pallas as pl
from jax.experimental.pallas import tpu as pltpu
from jax.experimental.pallas import tpu_sc as plsc

N = 10000
E = 320000
D = 128
H = 128
G3 = 3 * H

NC = 2
NS = 16
NW = NC * NS
CHUNK = 128
CPT = -(-E // (NW * CHUNK))
E_PAD = NW * CPT * CHUNK
N_ACC = 10240
RPT = N_ACC // NS
DUMMY = N

_MESH = dict(core_axis_name="c", subcore_axis_name="s", num_cores=NC,
             num_subcores=NS)


def _deg_body(dst_hbm, zeros_hbm, ones_hbm, out_hbm, dst_v, ones_v, deg_sh):
    c = lax.axis_index("c")
    s = lax.axis_index("s")
    wid = s * NC + c
    sl = pl.ds(s * RPT, RPT)
    pltpu.sync_copy(zeros_hbm.at[sl], deg_sh.at[sl])
    pltpu.sync_copy(ones_hbm, ones_v)
    pltpu.sync_copy(dst_hbm.at[wid], dst_v)
    plsc.subcore_barrier()

    def body(j, carry):
        pltpu.sync_copy(ones_v, deg_sh.at[dst_v.at[j]], add=True)
        return carry

    lax.fori_loop(0, CPT, body, 0)
    plsc.subcore_barrier()
    pltpu.sync_copy(deg_sh.at[sl], out_hbm.at[c, sl])


def _sc_deg(dst_r, zeros_deg, ones_chunk):
    fn = pl.kernel(
        _deg_body,
        out_type=jax.ShapeDtypeStruct((NC, N_ACC, D), jnp.float32),
        mesh=plsc.VectorSubcoreMesh(**_MESH),
        scratch_types=[
            pltpu.VMEM((CPT, CHUNK), jnp.int32),
            pltpu.VMEM((CHUNK, D), jnp.float32),
            pltpu.VMEM_SHARED((N_ACC, D), jnp.float32),
        ],
    )
    return fn(dst_r, zeros_deg, ones_chunk)


def _agg_body(s_hbm, src_hbm, dst_hbm, zeros_hbm, out_hbm,
              src_v, dst_v, rows, acc_sh, sem):
    c = lax.axis_index("c")
    s = lax.axis_index("s")
    wid = s * NC + c
    sl = pl.ds(s * RPT, RPT)
    pltpu.sync_copy(zeros_hbm.at[sl], acc_sh.at[sl])
    pltpu.sync_copy(src_hbm.at[wid], src_v)
    pltpu.sync_copy(dst_hbm.at[wid], dst_v)
    plsc.subcore_barrier()

    def body(j, carry):
        pltpu.async_copy(s_hbm.at[src_v.at[j]], rows, sem).wait()
        pltpu.sync_copy(rows, acc_sh.at[dst_v.at[j]], add=True)
        return carry

    lax.fori_loop(0, CPT, body, 0)
    plsc.subcore_barrier()
    pltpu.sync_copy(acc_sh.at[sl], out_hbm.at[c, sl])


def _sc_agg(s_rows, src_r, dst_r, zeros_acc):
    fn = pl.kernel(
        _agg_body,
        out_type=jax.ShapeDtypeStruct((NC, N_ACC, D), jnp.float32),
        mesh=plsc.VectorSubcoreMesh(**_MESH),
        scratch_types=[
            pltpu.VMEM((CPT, CHUNK), jnp.int32),
            pltpu.VMEM((CPT, CHUNK), jnp.int32),
            pltpu.VMEM((CHUNK, D), jnp.float32),
            pltpu.VMEM_SHARED((N_ACC, D), jnp.float32),
            pltpu.SemaphoreType.DMA,
        ],
    )
    return fn(s_rows, src_r, dst_r, zeros_acc)


R1 = 1000
R4 = 400


def _k1_body(x_ref, w_ref, dega_ref, degb_ref, s_ref, dinv_ref):
    dinv = lax.rsqrt(dega_ref[...] + degb_ref[...] + 1.0)
    u = jnp.dot(x_ref[...], w_ref[...], preferred_element_type=jnp.float32)
    s_ref[...] = u * dinv
    dinv_ref[...] = dinv


def _k1(x, W1, dega, degb):
    grid = N // R1
    return pl.pallas_call(
        _k1_body,
        grid=(grid,),
        in_specs=[
            pl.BlockSpec((R1, D), lambda i: (i, 0)),
            pl.BlockSpec((D, H), lambda i: (0, 0)),
            pl.BlockSpec((R1, 1), lambda i: (i, 0)),
            pl.BlockSpec((R1, 1), lambda i: (i, 0)),
        ],
        out_specs=[
            pl.BlockSpec((R1, H), lambda i: (i, 0)),
            pl.BlockSpec((R1, 1), lambda i: (i, 0)),
        ],
        out_shape=[
            jax.ShapeDtypeStruct((N, H), jnp.float32),
            jax.ShapeDtypeStruct((N, 1), jnp.float32),
        ],
    )(x, W1, dega, degb)


def _k2_body(aa_ref, ab_ref, s_ref, dinv_ref, b_ref, w_ref, out_ref):
    h1 = dinv_ref[...] * (aa_ref[...] + ab_ref[...] + s_ref[...]) + b_ref[...]
    u = jnp.dot(h1, w_ref[...], preferred_element_type=jnp.float32)
    out_ref[...] = dinv_ref[...] * u


def _k2(aa, ab, s1, dinv, b1, W2):
    grid = N // R1
    return pl.pallas_call(
        _k2_body,
        grid=(grid,),
        in_specs=[
            pl.BlockSpec((R1, H), lambda i: (i, 0)),
            pl.BlockSpec((R1, H), lambda i: (i, 0)),
            pl.BlockSpec((R1, H), lambda i: (i, 0)),
            pl.BlockSpec((R1, 1), lambda i: (i, 0)),
            pl.BlockSpec((1, H), lambda i: (0, 0)),
            pl.BlockSpec((H, H), lambda i: (0, 0)),
        ],
        out_specs=pl.BlockSpec((R1, H), lambda i: (i, 0)),
        out_shape=jax.ShapeDtypeStruct((N, H), jnp.float32),
    )(aa, ab, s1, dinv, b1, W2)


def _k3_body(aa_ref, ab_ref, s_ref, dinv_ref, b_ref, w_ref, bih_ref, out_ref):
    h2 = dinv_ref[...] * (aa_ref[...] + ab_ref[...] + s_ref[...]) + b_ref[...]
    u = jnp.dot(h2, w_ref[...], preferred_element_type=jnp.float32)
    out_ref[...] = u + bih_ref[...]


def _k3(aa, ab, s2, dinv, b2, WihT, bih):
    grid = N // R1
    return pl.pallas_call(
        _k3_body,
        grid=(grid,),
        in_specs=[
            pl.BlockSpec((R1, H), lambda i: (i, 0)),
            pl.BlockSpec((R1, H), lambda i: (i, 0)),
            pl.BlockSpec((R1, H), lambda i: (i, 0)),
            pl.BlockSpec((R1, 1), lambda i: (i, 0)),
            pl.BlockSpec((1, H), lambda i: (0, 0)),
            pl.BlockSpec((H, G3), lambda i: (0, 0)),
            pl.BlockSpec((1, G3), lambda i: (0, 0)),
        ],
        out_specs=pl.BlockSpec((R1, G3), lambda i: (i, 0)),
        out_shape=jax.ShapeDtypeStruct((N, G3), jnp.float32),
    )(aa, ab, s2, dinv, b2, WihT, bih)


def _k4_body(gi_ref, whhT_ref, bhh_ref, wfcT_ref, bfc_ref, y_ref,
             h_ref, hs_ref):
    @pl.when(pl.program_id(0) == 0)
    def _():
        h_ref[...] = jnp.zeros_like(h_ref)

    whhT = whhT_ref[...]
    bhh = bhh_ref[...]

    def step(i, h):
        gh = jnp.dot(h, whhT, preferred_element_type=jnp.float32) + bhh
        gi = gi_ref[pl.ds(i, 1), :]
        rz = jax.nn.sigmoid(gi[:, :2 * H] + gh[:, :2 * H])
        r = rz[:, :H]
        z = rz[:, H:]
        n = jnp.tanh(gi[:, 2 * H:] + r * gh[:, 2 * H:])
        hn = (1.0 - z) * n + z * h
        hs_ref[pl.ds(i, 1), :] = hn
        return hn

    h = lax.fori_loop(0, R4, step, h_ref[...])
    h_ref[...] = h
    y_ref[...] = (jnp.dot(hs_ref[...], wfcT_ref[...],
                          preferred_element_type=jnp.float32) + bfc_ref[...])


def _k4(gi, WhhT, bhh, WfcT, bfc):
    grid = N // R4
    return pl.pallas_call(
        _k4_body,
        grid=(grid,),
        in_specs=[
            pl.BlockSpec((R4, G3), lambda i: (i, 0)),
            pl.BlockSpec((H, G3), lambda i: (0, 0)),
            pl.BlockSpec((1, G3), lambda i: (0, 0)),
            pl.BlockSpec((H, 1), lambda i: (0, 0)),
            pl.BlockSpec((1, 1), lambda i: (0, 0)),
        ],
        out_specs=pl.BlockSpec((R4, 1), lambda i: (i, 0)),
        out_shape=jax.ShapeDtypeStruct((N, 1), jnp.float32),
        scratch_shapes=[
            pltpu.VMEM((1, H), jnp.float32),
            pltpu.VMEM((R4, H), jnp.float32),
        ],
    )(gi, WhhT, bhh, WfcT, bfc)


def kernel(x, edge_index, W1, b1, W2, b2, Wih, Whh, bih, bhh, Wfc, bfc):
    f32 = jnp.float32
    pad = E_PAD - E
    src_r = jnp.concatenate(
        [edge_index[0], jnp.zeros((pad,), jnp.int32)]).reshape(NW, CPT, CHUNK)
    dst_r = jnp.concatenate(
        [edge_index[1], jnp.full((pad,), DUMMY, jnp.int32)]).reshape(
            NW, CPT, CHUNK)
    zeros_acc = jnp.zeros((N_ACC, D), f32)
    ones_chunk = jnp.ones((CHUNK, D), f32)

    deg = _sc_deg(dst_r, zeros_acc, ones_chunk)
    dega = deg[0, :N, 0:1]
    degb = deg[1, :N, 0:1]

    s1, dinv = _k1(x, W1, dega, degb)
    agg1 = _sc_agg(s1, src_r, dst_r, zeros_acc)
    s2 = _k2(agg1[0, :N], agg1[1, :N], s1, dinv, b1.reshape(1, H), W2)
    agg2 = _sc_agg(s2, src_r, dst_r, zeros_acc)
    gi = _k3(agg2[0, :N], agg2[1, :N], s2, dinv, b2.reshape(1, H),
             Wih.T, bih.reshape(1, G3))
    y = _k4(gi, Whh.T, bhh.reshape(1, G3), Wfc.T, bfc.reshape(1, 1))
    return y
